# Initial kernel scaffold; baseline (speedup 1.0000x reference)
import functools

import jax
import jax.numpy as jnp
from jax import lax
from jax.experimental import pallas as pl
from jax.experimental.pallas import tpu as pltpu
from jax.experimental.pallas import tpu_sc as plsc

_NC = 2
_NS = 16
_NW = _NC * _NS


def _sc_gather(word_table, user_table, pos16, lang16,
               word_id, user_idx, pos_id, lang_id):
    B = word_id.shape[0]
    bpw = B // _NW
    mesh = plsc.VectorSubcoreMesh(core_axis_name="c", subcore_axis_name="s")
    f32 = jnp.float32

    @functools.partial(
        pl.kernel,
        out_type=[
            jax.ShapeDtypeStruct((B, 32), f32),
            jax.ShapeDtypeStruct((B, 32), f32),
            jax.ShapeDtypeStruct((B, 16), f32),
            jax.ShapeDtypeStruct((B, 16), f32),
        ],
        mesh=mesh,
        scratch_types=[
            pltpu.VMEM((bpw,), jnp.int32),
            pltpu.VMEM((bpw, 32), f32),
            pltpu.VMEM((bpw, 16), f32),
            pltpu.SemaphoreType.DMA,
        ],
    )
    def gather_kernel(word_hbm, user_hbm, pos_hbm, lang_hbm,
                      wid_hbm, uid_hbm, pid_hbm, lid_hbm,
                      wv_hbm, uv_hbm, pv_hbm, lv_hbm,
                      i_v, r32_v, r16_v, sem):
        wid = lax.axis_index("s") * _NC + lax.axis_index("c")
        base = wid * bpw
        for tbl, idx, out, rv in (
            (word_hbm, wid_hbm, wv_hbm, r32_v),
            (user_hbm, uid_hbm, uv_hbm, r32_v),
            (pos_hbm, pid_hbm, pv_hbm, r16_v),
            (lang_hbm, lid_hbm, lv_hbm, r16_v),
        ):
            pltpu.sync_copy(idx.at[pl.ds(base, bpw)], i_v)
            pltpu.async_copy(tbl.at[i_v], rv, sem).wait()
            pltpu.sync_copy(rv, out.at[pl.ds(base, bpw)])

    return gather_kernel(word_table, user_table, pos16, lang16,
                         word_id, user_idx, pos_id, lang_id)


def _mlp_body(wv, uv, pv, lv, nf, dt,
              w1w, w1u, w1p, w1l, w1n, b1, w2, b2,
              p_out, h_out):
    f32 = jnp.float32
    acc = jnp.dot(wv[...], w1w[...], preferred_element_type=f32)
    acc += jnp.dot(uv[...], w1u[...], preferred_element_type=f32)
    acc += jnp.dot(pv[...], w1p[...], preferred_element_type=f32)
    acc += jnp.dot(lv[...], w1l[...], preferred_element_type=f32)
    acc += jnp.dot(nf[...], w1n[...], preferred_element_type=f32)
    h1 = jnp.maximum(acc + b1[...], 0.0)
    dp = jnp.sum(h1 * w2[...], axis=1, keepdims=True) + b2[...]
    dp = jnp.clip(dp, -6.58, 8.1)
    h = jnp.clip(jnp.exp2(dp), 0.0104, 274.0)
    p = jnp.clip(jnp.exp2(-dt[...] / h), 0.0001, 0.9999)
    p_out[...] = p
    h_out[...] = h


def kernel(word_id, user_idx, pos_id, lang_id, num_features, delta_t,
           word_table, user_table, pos_table, lang_table, W1, b1, W2, b2):
    B = word_id.shape[0]
    f32 = jnp.float32

    pos16 = jnp.pad(pos_table, ((0, 0), (0, 8)))
    lang16 = jnp.pad(lang_table, ((0, 0), (0, 8)))
    wv, uv, pv, lv = _sc_gather(word_table, user_table, pos16, lang16,
                                word_id, user_idx, pos_id, lang_id)

    nf8 = jnp.pad(num_features, ((0, 0), (0, 3)))
    dt2 = delta_t.reshape(B, 1)
    w1w = W1[0:32]
    w1u = W1[32:64]
    w1p = jnp.pad(W1[64:72], ((0, 8), (0, 0)))
    w1l = jnp.pad(W1[72:80], ((0, 8), (0, 0)))
    w1n = jnp.pad(W1[80:85], ((0, 3), (0, 0)))
    b1r = b1.reshape(1, 64)
    w2r = W2.reshape(1, 64)
    b2r = b2.reshape(1, 1)

    p2, h2 = pl.pallas_call(
        _mlp_body,
        out_shape=[
            jax.ShapeDtypeStruct((B, 1), f32),
            jax.ShapeDtypeStruct((B, 1), f32),
        ],
    )(wv, uv, pv, lv, nf8, dt2, w1w, w1u, w1p, w1l, w1n, b1r, w2r, b2r)

    return p2.reshape(B), h2.reshape(B)


# speedup vs baseline: 1.9643x; 1.9643x over previous
import functools

import jax
import jax.numpy as jnp
from jax import lax
from jax.experimental import pallas as pl
from jax.experimental.pallas import tpu as pltpu
from jax.experimental.pallas import tpu_sc as plsc

_NC = 2
_NS = 16
_NW = _NC * _NS


def _sc_gather(word_table, user_table, pos16, lang16,
               word_id, user_idx, pos_id, lang_id):
    B = word_id.shape[0]
    bpw = B // _NW
    mesh = plsc.VectorSubcoreMesh(core_axis_name="c", subcore_axis_name="s")
    f32 = jnp.float32

    @functools.partial(
        pl.kernel,
        out_type=[
            jax.ShapeDtypeStruct((B, 32), f32),
            jax.ShapeDtypeStruct((B, 32), f32),
            jax.ShapeDtypeStruct((B, 16), f32),
            jax.ShapeDtypeStruct((B, 16), f32),
        ],
        mesh=mesh,
        scratch_types=[
            pltpu.VMEM((bpw,), jnp.int32),
            pltpu.VMEM((bpw, 32), f32),
            pltpu.VMEM((bpw, 16), f32),
            pltpu.SemaphoreType.DMA,
        ],
    )
    def gather_kernel(word_hbm, user_hbm, pos_hbm, lang_hbm,
                      wid_hbm, uid_hbm, pid_hbm, lid_hbm,
                      wv_hbm, uv_hbm, pv_hbm, lv_hbm,
                      i_v, r32_v, r16_v, sem):
        wid = lax.axis_index("s") * _NC + lax.axis_index("c")
        base = wid * bpw
        for tbl, idx, out, rv in (
            (word_hbm, wid_hbm, wv_hbm, r32_v),
            (user_hbm, uid_hbm, uv_hbm, r32_v),
            (pos_hbm, pid_hbm, pv_hbm, r16_v),
            (lang_hbm, lid_hbm, lv_hbm, r16_v),
        ):
            pltpu.sync_copy(idx.at[pl.ds(base, bpw)], i_v)
            pltpu.async_copy(tbl.at[i_v], rv, sem).wait()
            pltpu.sync_copy(rv, out.at[pl.ds(base, bpw)])

    return gather_kernel(word_table, user_table, pos16, lang16,
                         word_id, user_idx, pos_id, lang_id)


def _mlp_body(wv, uv, pv, lv, nf, dt,
              w1w, w1u, w1p, w1l, w1n, b1, w2, b2,
              p_out, h_out):
    f32 = jnp.float32
    acc = jnp.dot(wv[...], w1w[...], preferred_element_type=f32)
    acc += jnp.dot(uv[...], w1u[...], preferred_element_type=f32)
    acc += jnp.dot(pv[...], w1p[...], preferred_element_type=f32)
    acc += jnp.dot(lv[...], w1l[...], preferred_element_type=f32)
    acc += jnp.dot(nf[...], w1n[...], preferred_element_type=f32)
    h1 = jnp.maximum(acc + b1[...], 0.0)
    dp = jnp.sum(h1 * w2[...], axis=1, keepdims=True) + b2[...]
    dp = jnp.clip(dp, -6.58, 8.1)
    h = jnp.clip(jnp.exp2(dp), 0.0104, 274.0)
    p = jnp.clip(jnp.exp2(-dt[...] / h), 0.0001, 0.9999)
    p_out[...] = p
    h_out[...] = h


def kernel(word_id, user_idx, pos_id, lang_id, num_features, delta_t,
           word_table, user_table, pos_table, lang_table, W1, b1, W2, b2):
    B = word_id.shape[0]
    f32 = jnp.float32

    wv = jnp.take(word_table, word_id, axis=0)
    uv = jnp.take(user_table, user_idx, axis=0)
    pv = jnp.pad(jnp.take(pos_table, pos_id, axis=0), ((0, 0), (0, 8)))
    lv = jnp.pad(jnp.take(lang_table, lang_id, axis=0), ((0, 0), (0, 8)))

    nf8 = jnp.pad(num_features, ((0, 0), (0, 3)))
    dt2 = delta_t.reshape(B, 1)
    w1w = W1[0:32]
    w1u = W1[32:64]
    w1p = jnp.pad(W1[64:72], ((0, 8), (0, 0)))
    w1l = jnp.pad(W1[72:80], ((0, 8), (0, 0)))
    w1n = jnp.pad(W1[80:85], ((0, 3), (0, 0)))
    b1r = b1.reshape(1, 64)
    w2r = W2.reshape(1, 64)
    b2r = b2.reshape(1, 1)

    BLK = 2048
    row = lambda d: pl.BlockSpec((BLK, d), lambda i: (i, 0))
    full = lambda s: pl.BlockSpec(s, lambda i: (0, 0))
    p2, h2 = pl.pallas_call(
        _mlp_body,
        grid=(B // BLK,),
        in_specs=[
            row(32), row(32), row(16), row(16), row(8), row(1),
            full((32, 64)), full((32, 64)), full((16, 64)), full((16, 64)),
            full((8, 64)), full((1, 64)), full((1, 64)), full((1, 1)),
        ],
        out_specs=[row(1), row(1)],
        out_shape=[
            jax.ShapeDtypeStruct((B, 1), f32),
            jax.ShapeDtypeStruct((B, 1), f32),
        ],
    )(wv, uv, pv, lv, nf8, dt2, w1w, w1u, w1p, w1l, w1n, b1r, w2r, b2r)

    return p2.reshape(B), h2.reshape(B)
